# baseline (speedup 1.0000x reference)
import jax
import jax.numpy as jnp
from jax import lax
from jax.experimental import pallas as pl
from jax.experimental.pallas import tpu as pltpu
from jax.experimental.pallas import tpu_sc as plsc

_N = 10000
_NP = 10240
_E = 320000
_NC = 2
_NS = 16
_NW = _NC * _NS
_EPW = _E // _NW
_C = 80
_EPP = 10080
_NCHUNK = _EPP // _C
_RPT = _NP // _NS
_ZC = 128
_MB = 1000
_MG = _N // _MB


def _mm_block(x_ref, w_ref, o_ref):
    o_ref[...] = jnp.dot(x_ref[...], w_ref[...], preferred_element_type=jnp.float32)


def _mm_fused_block(pa_ref, pb_ref, b_ref, w_ref, o_ref):
    h = jnp.maximum(pa_ref[...] + pb_ref[...] + b_ref[...], 0.0)
    o_ref[...] = jnp.dot(h, w_ref[...], preferred_element_type=jnp.float32)


def _bias_add_block(pa_ref, pb_ref, b_ref, o_ref):
    m = o_ref.shape[1]
    o_ref[...] = pa_ref[:, :m] + pb_ref[:, :m] + b_ref[...]


def _mm(x, w):
    n, k = x.shape
    m = w.shape[1]
    return pl.pallas_call(
        _mm_block,
        grid=(_MG,),
        in_specs=[
            pl.BlockSpec((_MB, k), lambda i: (i, 0)),
            pl.BlockSpec((k, m), lambda i: (0, 0)),
        ],
        out_specs=pl.BlockSpec((_MB, m), lambda i: (i, 0)),
        out_shape=jax.ShapeDtypeStruct((n, m), jnp.float32),
    )(x, w)


def _mm_fused(pa, pb, b, w):
    k = pa.shape[1]
    n = _N
    m = w.shape[1]
    return pl.pallas_call(
        _mm_fused_block,
        grid=(_MG,),
        in_specs=[
            pl.BlockSpec((_MB, k), lambda i: (i, 0)),
            pl.BlockSpec((_MB, k), lambda i: (i, 0)),
            pl.BlockSpec((1, k), lambda i: (0, 0)),
            pl.BlockSpec((k, m), lambda i: (0, 0)),
        ],
        out_specs=pl.BlockSpec((_MB, m), lambda i: (i, 0)),
        out_shape=jax.ShapeDtypeStruct((n, m), jnp.float32),
    )(pa, pb, b, w)


def _bias_add(pa, pb, b):
    m = b.shape[1]
    n = _N
    return pl.pallas_call(
        _bias_add_block,
        grid=(_MG,),
        in_specs=[
            pl.BlockSpec((_MB, 128), lambda i: (i, 0)),
            pl.BlockSpec((_MB, 128), lambda i: (i, 0)),
            pl.BlockSpec((1, m), lambda i: (0, 0)),
        ],
        out_specs=pl.BlockSpec((_MB, m), lambda i: (i, 0)),
        out_shape=jax.ShapeDtypeStruct((n, m), jnp.float32),
    )(pa, pb, b)


def _spmm(edata, adjt, h2, d):
    nvec = d // 16
    mesh = plsc.VectorSubcoreMesh(core_axis_name="c", subcore_axis_name="s")

    def body(edata_hbm, adj_hbm, h2_hbm, out_hbm, meta_v, adj_v, dstb_v,
             rows_v, acc,
             gs0, gs1, gs2, ss0, ss1, ss2, ms0, ms1, ms2):
        gs = (gs0, gs1, gs2)
        ss = (ss0, ss1, ss2)
        ms = (ms0, ms1, ms2)
        cid = lax.axis_index("c")
        sid = lax.axis_index("s")
        wid = sid * _NC + cid

        zeros = jnp.zeros((16,), jnp.float32)

        def zrow(r, carry):
            for j in range(nvec):
                rows_v[0, r, pl.ds(j * 16, 16)] = zeros
            return carry

        lax.fori_loop(0, _C, zrow, 0)
        zsems = [gs0, gs1, gs2, ss0, ss1, ss2, gs0, gs1]
        for t in range(_RPT // _C):
            pltpu.async_copy(rows_v.at[0],
                             acc.at[pl.ds(sid * _RPT + t * _C, _C)], zsems[t])
        for t in range(_RPT // _C):
            pltpu.make_async_copy(rows_v.at[0],
                                  acc.at[pl.ds(sid * _RPT + t * _C, _C)],
                                  zsems[t]).wait()
        plsc.subcore_barrier()

        def mstage(k, m):
            pltpu.async_copy(edata_hbm.at[wid, k], meta_v.at[m], ms[m])
            pltpu.async_copy(adj_hbm.at[wid, k], adj_v.at[m], ms[m])

        def mwait(m):
            pltpu.make_async_copy(edata_hbm.at[wid, 0], meta_v.at[m],
                                  ms[m]).wait()
            pltpu.make_async_copy(adj_hbm.at[wid, 0], adj_v.at[m],
                                  ms[m]).wait()

        def gissue(s):
            pltpu.async_copy(h2_hbm.at[meta_v.at[s, 0]], rows_v.at[s], gs[s])

        def gwait(s):
            pltpu.make_async_copy(h2_hbm.at[meta_v.at[s, 0]], rows_v.at[s],
                                  gs[s]).wait()

        def sissue(s):
            pltpu.async_copy(rows_v.at[s], acc.at[dstb_v.at[s]], ss[s],
                             add=True)

        def swait(s):
            pltpu.make_async_copy(rows_v.at[s], acc.at[dstb_v.at[s]],
                                  ss[s]).wait()

        def privcopy(s):
            for t in range(_C // 16):
                sl = pl.ds(t * 16, 16)
                dstb_v[s, sl] = meta_v[s, 1, sl]

        def compute(s):
            def egroup(g, c2):
                w16 = adj_v[s, pl.ds(g * 16, 16)]
                for i in range(16):
                    w = jnp.broadcast_to(w16[i], (16,))
                    for j in range(nvec):
                        sl = pl.ds(j * 16, 16)
                        rows_v[s, g * 16 + i, sl] = rows_v[s, g * 16 + i, sl] * w
                return c2

            lax.fori_loop(0, _C // 16, egroup, 0)

        mstage(0, 0)
        mstage(1, 1)
        mstage(2, 2)
        mwait(0)
        gissue(0)
        mwait(1)
        gissue(1)

        def pipe(gi, carry):
            for j in range(3):
                k = gi * 3 + j
                s = j
                s2 = (j + 2) % 3

                gwait(s)
                privcopy(s)
                compute(s)
                sissue(s)

                @pl.when(k + 3 < _NCHUNK)
                def _():
                    mstage(k + 3, s)

                @pl.when(k + 2 < _NCHUNK)
                def _():
                    @pl.when(k >= 1)
                    def _():
                        swait(s2)

                    mwait(s2)
                    gissue(s2)
            return carry

        lax.fori_loop(0, _NCHUNK // 3, pipe, 0)
        swait(0)
        swait(1)
        swait(2)
        plsc.subcore_barrier()
        pltpu.sync_copy(acc.at[pl.ds(sid * _RPT, _RPT)],
                        out_hbm.at[cid, pl.ds(sid * _RPT, _RPT)])

    f = pl.kernel(
        body,
        out_type=jax.ShapeDtypeStruct((_NC, _NP, d), jnp.float32),
        mesh=mesh,
        scratch_types=[
            pltpu.VMEM((3, 2, _C), jnp.int32),
            pltpu.VMEM((3, _C), jnp.float32),
            pltpu.VMEM((3, _C), jnp.int32),
            pltpu.VMEM((3, _C, d), jnp.float32),
            pltpu.VMEM_SHARED((_NP, d), jnp.float32),
        ] + [pltpu.SemaphoreType.DMA] * 9,
    )
    return f(edata, adjt, h2)


def kernel(x, adj, edge_index, isVal, W0, b0, W1, b1, W2, b2, W_out, b_out):
    npad = _EPP - _EPW
    pad_src = jnp.broadcast_to(jnp.arange(npad, dtype=jnp.int32), (_NW, npad))
    pad_dst = jnp.broadcast_to(jnp.arange(_N, _N + npad, dtype=jnp.int32),
                               (_NW, npad))
    src3 = jnp.concatenate([edge_index[0].reshape(_NW, _EPW), pad_src],
                           axis=1).reshape(_NW, _NCHUNK, _C)
    dst3 = jnp.concatenate([edge_index[1].reshape(_NW, _EPW), pad_dst],
                           axis=1).reshape(_NW, _NCHUNK, _C)
    edata = jnp.stack([src3, dst3], axis=2)
    adjt = jnp.pad(adj.reshape(_NW, _EPW), ((0, 0), (0, npad))).reshape(
        _NW, _NCHUNK, _C)

    h2 = _mm(x, W0)
    p = _spmm(edata, adjt, h2, 128)
    h2 = _mm_fused(p[0], p[1], b0.reshape(1, -1), W1)
    p = _spmm(edata, adjt, h2, 128)
    h2 = _mm_fused(p[0], p[1], b1.reshape(1, -1), W2)
    p = _spmm(edata, adjt, h2, 128)
    W_out_p = jnp.zeros((W_out.shape[0], 128), jnp.float32).at[:, :W_out.shape[1]].set(W_out)
    h2 = _mm_fused(p[0], p[1], b2.reshape(1, -1), W_out_p)
    p = _spmm(edata, adjt, h2, 128)
    return _bias_add(p[0], p[1], b_out.reshape(1, -1))


# speedup vs baseline: 1.0267x; 1.0267x over previous
import jax
import jax.numpy as jnp
from jax import lax
from jax.experimental import pallas as pl
from jax.experimental.pallas import tpu as pltpu
from jax.experimental.pallas import tpu_sc as plsc

_N = 10000
_NP = 10240
_E = 320000
_NC = 2
_NS = 16
_NW = _NC * _NS
_EPW = _E // _NW
_C = 96
_EPP = 10080
_NCHUNK = _EPP // _C
_ZR = 80
_RPT = _NP // _NS
_ZC = 128
_MB = 1000
_MG = _N // _MB


def _mm_block(x_ref, w_ref, o_ref):
    o_ref[...] = jnp.dot(x_ref[...], w_ref[...], preferred_element_type=jnp.float32)


def _mm_fused_block(pa_ref, pb_ref, b_ref, w_ref, o_ref):
    h = jnp.maximum(pa_ref[...] + pb_ref[...] + b_ref[...], 0.0)
    o_ref[...] = jnp.dot(h, w_ref[...], preferred_element_type=jnp.float32)


def _bias_add_block(pa_ref, pb_ref, b_ref, o_ref):
    m = o_ref.shape[1]
    o_ref[...] = pa_ref[:, :m] + pb_ref[:, :m] + b_ref[...]


def _mm(x, w):
    n, k = x.shape
    m = w.shape[1]
    return pl.pallas_call(
        _mm_block,
        grid=(_MG,),
        in_specs=[
            pl.BlockSpec((_MB, k), lambda i: (i, 0)),
            pl.BlockSpec((k, m), lambda i: (0, 0)),
        ],
        out_specs=pl.BlockSpec((_MB, m), lambda i: (i, 0)),
        out_shape=jax.ShapeDtypeStruct((n, m), jnp.float32),
    )(x, w)


def _mm_fused(pa, pb, b, w):
    k = pa.shape[1]
    n = _N
    m = w.shape[1]
    return pl.pallas_call(
        _mm_fused_block,
        grid=(_MG,),
        in_specs=[
            pl.BlockSpec((_MB, k), lambda i: (i, 0)),
            pl.BlockSpec((_MB, k), lambda i: (i, 0)),
            pl.BlockSpec((1, k), lambda i: (0, 0)),
            pl.BlockSpec((k, m), lambda i: (0, 0)),
        ],
        out_specs=pl.BlockSpec((_MB, m), lambda i: (i, 0)),
        out_shape=jax.ShapeDtypeStruct((n, m), jnp.float32),
    )(pa, pb, b, w)


def _bias_add(pa, pb, b):
    m = b.shape[1]
    n = _N
    return pl.pallas_call(
        _bias_add_block,
        grid=(_MG,),
        in_specs=[
            pl.BlockSpec((_MB, 128), lambda i: (i, 0)),
            pl.BlockSpec((_MB, 128), lambda i: (i, 0)),
            pl.BlockSpec((1, m), lambda i: (0, 0)),
        ],
        out_specs=pl.BlockSpec((_MB, m), lambda i: (i, 0)),
        out_shape=jax.ShapeDtypeStruct((n, m), jnp.float32),
    )(pa, pb, b)


def _spmm(edata, adjt, h2, d):
    nvec = d // 16
    mesh = plsc.VectorSubcoreMesh(core_axis_name="c", subcore_axis_name="s")

    def body(edata_hbm, adj_hbm, h2_hbm, out_hbm, meta_v, adj_v, dstb_v,
             rows_v, acc,
             gs0, gs1, gs2, ss0, ss1, ss2, ms0, ms1, ms2):
        gs = (gs0, gs1, gs2)
        ss = (ss0, ss1, ss2)
        ms = (ms0, ms1, ms2)
        cid = lax.axis_index("c")
        sid = lax.axis_index("s")
        wid = sid * _NC + cid

        zeros = jnp.zeros((16,), jnp.float32)

        def zrow(r, carry):
            for j in range(nvec):
                rows_v[0, r, pl.ds(j * 16, 16)] = zeros
            return carry

        lax.fori_loop(0, _ZR, zrow, 0)
        zsems = [gs0, gs1, gs2, ss0, ss1, ss2, gs0, gs1]
        for t in range(_RPT // _ZR):
            pltpu.async_copy(rows_v.at[0, pl.ds(0, _ZR)],
                             acc.at[pl.ds(sid * _RPT + t * _ZR, _ZR)], zsems[t])
        for t in range(_RPT // _ZR):
            pltpu.make_async_copy(rows_v.at[0, pl.ds(0, _ZR)],
                                  acc.at[pl.ds(sid * _RPT + t * _ZR, _ZR)],
                                  zsems[t]).wait()
        plsc.subcore_barrier()

        def mstage(k, m):
            pltpu.async_copy(edata_hbm.at[wid, k], meta_v.at[m], ms[m])
            pltpu.async_copy(adj_hbm.at[wid, k], adj_v.at[m], ms[m])

        def mwait(m):
            pltpu.make_async_copy(edata_hbm.at[wid, 0], meta_v.at[m],
                                  ms[m]).wait()
            pltpu.make_async_copy(adj_hbm.at[wid, 0], adj_v.at[m],
                                  ms[m]).wait()

        def gissue(s):
            pltpu.async_copy(h2_hbm.at[meta_v.at[s, 0]], rows_v.at[s], gs[s])

        def gwait(s):
            pltpu.make_async_copy(h2_hbm.at[meta_v.at[s, 0]], rows_v.at[s],
                                  gs[s]).wait()

        def sissue(s):
            pltpu.async_copy(rows_v.at[s], acc.at[dstb_v.at[s]], ss[s],
                             add=True)

        def swait(s):
            pltpu.make_async_copy(rows_v.at[s], acc.at[dstb_v.at[s]],
                                  ss[s]).wait()

        def privcopy(s):
            for t in range(_C // 16):
                sl = pl.ds(t * 16, 16)
                dstb_v[s, sl] = meta_v[s, 1, sl]

        def compute(s):
            def egroup(g, c2):
                w16 = adj_v[s, pl.ds(g * 16, 16)]
                for i in range(16):
                    w = jnp.broadcast_to(w16[i], (16,))
                    for j in range(nvec):
                        sl = pl.ds(j * 16, 16)
                        rows_v[s, g * 16 + i, sl] = rows_v[s, g * 16 + i, sl] * w
                return c2

            lax.fori_loop(0, _C // 16, egroup, 0)

        mstage(0, 0)
        mstage(1, 1)
        mstage(2, 2)
        mwait(0)
        gissue(0)
        mwait(1)
        gissue(1)

        def pipe(gi, carry):
            for j in range(3):
                k = gi * 3 + j
                s = j
                s2 = (j + 2) % 3

                gwait(s)
                privcopy(s)
                compute(s)
                sissue(s)

                @pl.when(k + 3 < _NCHUNK)
                def _():
                    mstage(k + 3, s)

                @pl.when(k + 2 < _NCHUNK)
                def _():
                    @pl.when(k >= 1)
                    def _():
                        swait(s2)

                    mwait(s2)
                    gissue(s2)
            return carry

        lax.fori_loop(0, _NCHUNK // 3, pipe, 0)
        swait(0)
        swait(1)
        swait(2)
        plsc.subcore_barrier()
        pltpu.sync_copy(acc.at[pl.ds(sid * _RPT, _RPT)],
                        out_hbm.at[cid, pl.ds(sid * _RPT, _RPT)])

    f = pl.kernel(
        body,
        out_type=jax.ShapeDtypeStruct((_NC, _NP, d), jnp.float32),
        mesh=mesh,
        scratch_types=[
            pltpu.VMEM((3, 2, _C), jnp.int32),
            pltpu.VMEM((3, _C), jnp.float32),
            pltpu.VMEM((3, _C), jnp.int32),
            pltpu.VMEM((3, _C, d), jnp.float32),
            pltpu.VMEM_SHARED((_NP, d), jnp.float32),
        ] + [pltpu.SemaphoreType.DMA] * 9,
    )
    return f(edata, adjt, h2)


def kernel(x, adj, edge_index, isVal, W0, b0, W1, b1, W2, b2, W_out, b_out):
    npad = _EPP - _EPW
    pad_src = jnp.broadcast_to(jnp.arange(npad, dtype=jnp.int32), (_NW, npad))
    pad_dst = jnp.broadcast_to(jnp.arange(_N, _N + npad, dtype=jnp.int32),
                               (_NW, npad))
    src3 = jnp.concatenate([edge_index[0].reshape(_NW, _EPW), pad_src],
                           axis=1).reshape(_NW, _NCHUNK, _C)
    dst3 = jnp.concatenate([edge_index[1].reshape(_NW, _EPW), pad_dst],
                           axis=1).reshape(_NW, _NCHUNK, _C)
    edata = jnp.stack([src3, dst3], axis=2)
    adjt = jnp.pad(adj.reshape(_NW, _EPW), ((0, 0), (0, npad))).reshape(
        _NW, _NCHUNK, _C)

    h2 = _mm(x, W0)
    p = _spmm(edata, adjt, h2, 128)
    h2 = _mm_fused(p[0], p[1], b0.reshape(1, -1), W1)
    p = _spmm(edata, adjt, h2, 128)
    h2 = _mm_fused(p[0], p[1], b1.reshape(1, -1), W2)
    p = _spmm(edata, adjt, h2, 128)
    W_out_p = jnp.zeros((W_out.shape[0], 128), jnp.float32).at[:, :W_out.shape[1]].set(W_out)
    h2 = _mm_fused(p[0], p[1], b2.reshape(1, -1), W_out_p)
    p = _spmm(edata, adjt, h2, 128)
    return _bias_add(p[0], p[1], b_out.reshape(1, -1))


# speedup vs baseline: 1.0419x; 1.0148x over previous
import jax
import jax.numpy as jnp
from jax import lax
from jax.experimental import pallas as pl
from jax.experimental.pallas import tpu as pltpu
from jax.experimental.pallas import tpu_sc as plsc

_N = 10000
_NP = 10240
_E = 320000
_NC = 2
_NS = 16
_NW = _NC * _NS
_EPW = _E // _NW
_C = 112
_EPP = 10080
_NCHUNK = _EPP // _C
_ZR = 80
_RPT = _NP // _NS
_ZC = 128
_MB = 1000
_MG = _N // _MB


def _mm_block(x_ref, w_ref, o_ref):
    o_ref[...] = jnp.dot(x_ref[...], w_ref[...], preferred_element_type=jnp.float32)


def _mm_fused_block(pa_ref, pb_ref, b_ref, w_ref, o_ref):
    h = jnp.maximum(pa_ref[...] + pb_ref[...] + b_ref[...], 0.0)
    o_ref[...] = jnp.dot(h, w_ref[...], preferred_element_type=jnp.float32)


def _bias_add_block(pa_ref, pb_ref, b_ref, o_ref):
    m = o_ref.shape[1]
    o_ref[...] = pa_ref[:, :m] + pb_ref[:, :m] + b_ref[...]


def _mm(x, w):
    n, k = x.shape
    m = w.shape[1]
    return pl.pallas_call(
        _mm_block,
        grid=(_MG,),
        in_specs=[
            pl.BlockSpec((_MB, k), lambda i: (i, 0)),
            pl.BlockSpec((k, m), lambda i: (0, 0)),
        ],
        out_specs=pl.BlockSpec((_MB, m), lambda i: (i, 0)),
        out_shape=jax.ShapeDtypeStruct((n, m), jnp.float32),
    )(x, w)


def _mm_fused(pa, pb, b, w):
    k = pa.shape[1]
    n = _N
    m = w.shape[1]
    return pl.pallas_call(
        _mm_fused_block,
        grid=(_MG,),
        in_specs=[
            pl.BlockSpec((_MB, k), lambda i: (i, 0)),
            pl.BlockSpec((_MB, k), lambda i: (i, 0)),
            pl.BlockSpec((1, k), lambda i: (0, 0)),
            pl.BlockSpec((k, m), lambda i: (0, 0)),
        ],
        out_specs=pl.BlockSpec((_MB, m), lambda i: (i, 0)),
        out_shape=jax.ShapeDtypeStruct((n, m), jnp.float32),
    )(pa, pb, b, w)


def _bias_add(pa, pb, b):
    m = b.shape[1]
    n = _N
    return pl.pallas_call(
        _bias_add_block,
        grid=(_MG,),
        in_specs=[
            pl.BlockSpec((_MB, 128), lambda i: (i, 0)),
            pl.BlockSpec((_MB, 128), lambda i: (i, 0)),
            pl.BlockSpec((1, m), lambda i: (0, 0)),
        ],
        out_specs=pl.BlockSpec((_MB, m), lambda i: (i, 0)),
        out_shape=jax.ShapeDtypeStruct((n, m), jnp.float32),
    )(pa, pb, b)


def _spmm(edata, adjt, h2, d):
    nvec = d // 16
    mesh = plsc.VectorSubcoreMesh(core_axis_name="c", subcore_axis_name="s")

    def body(edata_hbm, adj_hbm, h2_hbm, out_hbm, meta_v, adj_v, dstb_v,
             rows_v, acc,
             gs0, gs1, gs2, ss0, ss1, ss2, ms0, ms1, ms2):
        gs = (gs0, gs1, gs2)
        ss = (ss0, ss1, ss2)
        ms = (ms0, ms1, ms2)
        cid = lax.axis_index("c")
        sid = lax.axis_index("s")
        wid = sid * _NC + cid

        zeros = jnp.zeros((16,), jnp.float32)

        def zrow(r, carry):
            for j in range(nvec):
                rows_v[0, r, pl.ds(j * 16, 16)] = zeros
            return carry

        lax.fori_loop(0, _ZR, zrow, 0)
        zsems = [gs0, gs1, gs2, ss0, ss1, ss2, gs0, gs1]
        for t in range(_RPT // _ZR):
            pltpu.async_copy(rows_v.at[0, pl.ds(0, _ZR)],
                             acc.at[pl.ds(sid * _RPT + t * _ZR, _ZR)], zsems[t])
        for t in range(_RPT // _ZR):
            pltpu.make_async_copy(rows_v.at[0, pl.ds(0, _ZR)],
                                  acc.at[pl.ds(sid * _RPT + t * _ZR, _ZR)],
                                  zsems[t]).wait()
        plsc.subcore_barrier()

        def mstage(k, m):
            pltpu.async_copy(edata_hbm.at[wid, k], meta_v.at[m], ms[m])
            pltpu.async_copy(adj_hbm.at[wid, k], adj_v.at[m], ms[m])

        def mwait(m):
            pltpu.make_async_copy(edata_hbm.at[wid, 0], meta_v.at[m],
                                  ms[m]).wait()
            pltpu.make_async_copy(adj_hbm.at[wid, 0], adj_v.at[m],
                                  ms[m]).wait()

        def gissue(s):
            pltpu.async_copy(h2_hbm.at[meta_v.at[s, 0]], rows_v.at[s], gs[s])

        def gwait(s):
            pltpu.make_async_copy(h2_hbm.at[meta_v.at[s, 0]], rows_v.at[s],
                                  gs[s]).wait()

        def sissue(s):
            pltpu.async_copy(rows_v.at[s], acc.at[dstb_v.at[s]], ss[s],
                             add=True)

        def swait(s):
            pltpu.make_async_copy(rows_v.at[s], acc.at[dstb_v.at[s]],
                                  ss[s]).wait()

        def privcopy(s):
            for t in range(_C // 16):
                sl = pl.ds(t * 16, 16)
                dstb_v[s, sl] = meta_v[s, 1, sl]

        def compute(s):
            def egroup(g, c2):
                w16 = adj_v[s, pl.ds(g * 16, 16)]
                for i in range(16):
                    w = jnp.broadcast_to(w16[i], (16,))
                    for j in range(nvec):
                        sl = pl.ds(j * 16, 16)
                        rows_v[s, g * 16 + i, sl] = rows_v[s, g * 16 + i, sl] * w
                return c2

            lax.fori_loop(0, _C // 16, egroup, 0)

        mstage(0, 0)
        mstage(1, 1)
        mstage(2, 2)
        mwait(0)
        gissue(0)
        mwait(1)
        gissue(1)

        def pipe(gi, carry):
            for j in range(3):
                k = gi * 3 + j
                s = j
                s2 = (j + 2) % 3

                gwait(s)
                privcopy(s)
                compute(s)
                sissue(s)

                @pl.when(k + 3 < _NCHUNK)
                def _():
                    mstage(k + 3, s)

                @pl.when(k + 2 < _NCHUNK)
                def _():
                    @pl.when(k >= 1)
                    def _():
                        swait(s2)

                    mwait(s2)
                    gissue(s2)
            return carry

        lax.fori_loop(0, _NCHUNK // 3, pipe, 0)
        swait(0)
        swait(1)
        swait(2)
        plsc.subcore_barrier()
        pltpu.sync_copy(acc.at[pl.ds(sid * _RPT, _RPT)],
                        out_hbm.at[cid, pl.ds(sid * _RPT, _RPT)])

    f = pl.kernel(
        body,
        out_type=jax.ShapeDtypeStruct((_NC, _NP, d), jnp.float32),
        mesh=mesh,
        scratch_types=[
            pltpu.VMEM((3, 2, _C), jnp.int32),
            pltpu.VMEM((3, _C), jnp.float32),
            pltpu.VMEM((3, _C), jnp.int32),
            pltpu.VMEM((3, _C, d), jnp.float32),
            pltpu.VMEM_SHARED((_NP, d), jnp.float32),
        ] + [pltpu.SemaphoreType.DMA] * 9,
    )
    return f(edata, adjt, h2)


def kernel(x, adj, edge_index, isVal, W0, b0, W1, b1, W2, b2, W_out, b_out):
    npad = _EPP - _EPW
    pad_src = jnp.broadcast_to(jnp.arange(npad, dtype=jnp.int32), (_NW, npad))
    pad_dst = jnp.broadcast_to(jnp.arange(_N, _N + npad, dtype=jnp.int32),
                               (_NW, npad))
    src3 = jnp.concatenate([edge_index[0].reshape(_NW, _EPW), pad_src],
                           axis=1).reshape(_NW, _NCHUNK, _C)
    dst3 = jnp.concatenate([edge_index[1].reshape(_NW, _EPW), pad_dst],
                           axis=1).reshape(_NW, _NCHUNK, _C)
    edata = jnp.stack([src3, dst3], axis=2)
    adjt = jnp.pad(adj.reshape(_NW, _EPW), ((0, 0), (0, npad))).reshape(
        _NW, _NCHUNK, _C)

    h2 = _mm(x, W0)
    p = _spmm(edata, adjt, h2, 128)
    h2 = _mm_fused(p[0], p[1], b0.reshape(1, -1), W1)
    p = _spmm(edata, adjt, h2, 128)
    h2 = _mm_fused(p[0], p[1], b1.reshape(1, -1), W2)
    p = _spmm(edata, adjt, h2, 128)
    W_out_p = jnp.zeros((W_out.shape[0], 128), jnp.float32).at[:, :W_out.shape[1]].set(W_out)
    h2 = _mm_fused(p[0], p[1], b2.reshape(1, -1), W_out_p)
    p = _spmm(edata, adjt, h2, 128)
    return _bias_add(p[0], p[1], b_out.reshape(1, -1))


# speedup vs baseline: 1.0553x; 1.0129x over previous
import jax
import jax.numpy as jnp
from jax import lax
from jax.experimental import pallas as pl
from jax.experimental.pallas import tpu as pltpu
from jax.experimental.pallas import tpu_sc as plsc

_N = 10000
_NP = 10240
_E = 320000
_NC = 2
_NS = 16
_NW = _NC * _NS
_EPW = _E // _NW
_C = 112
_EPP = 10080
_NCHUNK = _EPP // _C
_ZR = 80
_RPT = _NP // _NS
_ZC = 128
_MB = 1000
_MG = _N // _MB


def _mm_block(x_ref, w_ref, o_ref):
    o_ref[...] = jnp.dot(x_ref[...], w_ref[...], preferred_element_type=jnp.float32)


def _mm_fused_block(pa_ref, pb_ref, b_ref, w_ref, o_ref):
    h = jnp.maximum(pa_ref[...] + pb_ref[...] + b_ref[...], 0.0)
    o_ref[...] = jnp.dot(h, w_ref[...], preferred_element_type=jnp.float32)


def _bias_add_block(pa_ref, pb_ref, b_ref, o_ref):
    m = o_ref.shape[1]
    o_ref[...] = pa_ref[:, :m] + pb_ref[:, :m] + b_ref[...]


def _mm(x, w):
    n, k = x.shape
    m = w.shape[1]
    return pl.pallas_call(
        _mm_block,
        grid=(_MG,),
        in_specs=[
            pl.BlockSpec((_MB, k), lambda i: (i, 0)),
            pl.BlockSpec((k, m), lambda i: (0, 0)),
        ],
        out_specs=pl.BlockSpec((_MB, m), lambda i: (i, 0)),
        out_shape=jax.ShapeDtypeStruct((n, m), jnp.float32),
    )(x, w)


def _mm_fused(pa, pb, b, w):
    k = pa.shape[1]
    n = _N
    m = w.shape[1]
    return pl.pallas_call(
        _mm_fused_block,
        grid=(_MG,),
        in_specs=[
            pl.BlockSpec((_MB, k), lambda i: (i, 0)),
            pl.BlockSpec((_MB, k), lambda i: (i, 0)),
            pl.BlockSpec((1, k), lambda i: (0, 0)),
            pl.BlockSpec((k, m), lambda i: (0, 0)),
        ],
        out_specs=pl.BlockSpec((_MB, m), lambda i: (i, 0)),
        out_shape=jax.ShapeDtypeStruct((n, m), jnp.float32),
    )(pa, pb, b, w)


def _bias_add(pa, pb, b):
    m = b.shape[1]
    n = _N
    return pl.pallas_call(
        _bias_add_block,
        grid=(_MG,),
        in_specs=[
            pl.BlockSpec((_MB, 128), lambda i: (i, 0)),
            pl.BlockSpec((_MB, 128), lambda i: (i, 0)),
            pl.BlockSpec((1, m), lambda i: (0, 0)),
        ],
        out_specs=pl.BlockSpec((_MB, m), lambda i: (i, 0)),
        out_shape=jax.ShapeDtypeStruct((n, m), jnp.float32),
    )(pa, pb, b)


def _spmm(edata, adjt, h2, d):
    nvec = d // 16
    mesh = plsc.VectorSubcoreMesh(core_axis_name="c", subcore_axis_name="s")

    def body(edata_hbm, adj_hbm, h2_hbm, out_hbm, meta_v, adj_v, dstb_v,
             rows_v, acc,
             gs0, gs1, gs2, ss0, ss1, ss2, ms0, ms1, ms2):
        gs = (gs0, gs1, gs2)
        ss = (ss0, ss1, ss2)
        ms = (ms0, ms1, ms2)
        cid = lax.axis_index("c")
        sid = lax.axis_index("s")
        wid = sid * _NC + cid

        def mstage(k, m):
            pltpu.async_copy(edata_hbm.at[wid, k], meta_v.at[m], ms[m])
            pltpu.async_copy(adj_hbm.at[wid, k], adj_v.at[m], ms[m])

        def mwait(m):
            pltpu.make_async_copy(edata_hbm.at[wid, 0], meta_v.at[m],
                                  ms[m]).wait()
            pltpu.make_async_copy(adj_hbm.at[wid, 0], adj_v.at[m],
                                  ms[m]).wait()

        def gissue(s):
            pltpu.async_copy(h2_hbm.at[meta_v.at[s, 0]], rows_v.at[s], gs[s])

        def gwait(s):
            pltpu.make_async_copy(h2_hbm.at[meta_v.at[s, 0]], rows_v.at[s],
                                  gs[s]).wait()

        def sissue(s):
            pltpu.async_copy(rows_v.at[s], acc.at[dstb_v.at[s]], ss[s],
                             add=True)

        def swait(s):
            pltpu.make_async_copy(rows_v.at[s], acc.at[dstb_v.at[s]],
                                  ss[s]).wait()

        def privcopy(s):
            for t in range(_C // 16):
                sl = pl.ds(t * 16, 16)
                dstb_v[s, sl] = meta_v[s, 1, sl]

        def compute(s):
            def egroup(g, c2):
                w16 = adj_v[s, pl.ds(g * 16, 16)]
                for i in range(16):
                    w = jnp.broadcast_to(w16[i], (16,))
                    for j in range(nvec):
                        sl = pl.ds(j * 16, 16)
                        rows_v[s, g * 16 + i, sl] = rows_v[s, g * 16 + i, sl] * w
                return c2

            lax.fori_loop(0, _C // 16, egroup, 0)

        mstage(0, 0)
        mstage(1, 1)
        mstage(2, 2)
        zeros = jnp.zeros((16,), jnp.float32)

        def zrow(r, carry):
            for j in range(nvec):
                rows_v[2, r, pl.ds(j * 16, 16)] = zeros
            return carry

        lax.fori_loop(0, _ZR, zrow, 0)
        mwait(0)
        gissue(0)
        mwait(1)
        gissue(1)
        zsems = [ss0, ss1, ss2, ss0, ss1, ss2, ss0, ss1]
        for t in range(_RPT // _ZR):
            pltpu.async_copy(rows_v.at[2, pl.ds(0, _ZR)],
                             acc.at[pl.ds(sid * _RPT + t * _ZR, _ZR)], zsems[t])
        for t in range(_RPT // _ZR):
            pltpu.make_async_copy(rows_v.at[2, pl.ds(0, _ZR)],
                                  acc.at[pl.ds(sid * _RPT + t * _ZR, _ZR)],
                                  zsems[t]).wait()
        plsc.subcore_barrier()


        def pipe(gi, carry):
            for j in range(3):
                k = gi * 3 + j
                s = j
                s2 = (j + 2) % 3

                gwait(s)
                privcopy(s)
                compute(s)
                sissue(s)

                @pl.when(k + 3 < _NCHUNK)
                def _():
                    mstage(k + 3, s)

                @pl.when(k + 2 < _NCHUNK)
                def _():
                    @pl.when(k >= 1)
                    def _():
                        swait(s2)

                    mwait(s2)
                    gissue(s2)
            return carry

        lax.fori_loop(0, _NCHUNK // 3, pipe, 0)
        swait(0)
        swait(1)
        swait(2)
        plsc.subcore_barrier()
        pltpu.sync_copy(acc.at[pl.ds(sid * _RPT, _RPT)],
                        out_hbm.at[cid, pl.ds(sid * _RPT, _RPT)])

    f = pl.kernel(
        body,
        out_type=jax.ShapeDtypeStruct((_NC, _NP, d), jnp.float32),
        mesh=mesh,
        scratch_types=[
            pltpu.VMEM((3, 2, _C), jnp.int32),
            pltpu.VMEM((3, _C), jnp.float32),
            pltpu.VMEM((3, _C), jnp.int32),
            pltpu.VMEM((3, _C, d), jnp.float32),
            pltpu.VMEM_SHARED((_NP, d), jnp.float32),
        ] + [pltpu.SemaphoreType.DMA] * 9,
    )
    return f(edata, adjt, h2)


def kernel(x, adj, edge_index, isVal, W0, b0, W1, b1, W2, b2, W_out, b_out):
    npad = _EPP - _EPW
    pad_src = jnp.broadcast_to(jnp.arange(npad, dtype=jnp.int32), (_NW, npad))
    pad_dst = jnp.broadcast_to(jnp.arange(_N, _N + npad, dtype=jnp.int32),
                               (_NW, npad))
    src3 = jnp.concatenate([edge_index[0].reshape(_NW, _EPW), pad_src],
                           axis=1).reshape(_NW, _NCHUNK, _C)
    dst3 = jnp.concatenate([edge_index[1].reshape(_NW, _EPW), pad_dst],
                           axis=1).reshape(_NW, _NCHUNK, _C)
    edata = jnp.stack([src3, dst3], axis=2)
    adjt = jnp.pad(adj.reshape(_NW, _EPW), ((0, 0), (0, npad))).reshape(
        _NW, _NCHUNK, _C)

    h2 = _mm(x, W0)
    p = _spmm(edata, adjt, h2, 128)
    h2 = _mm_fused(p[0], p[1], b0.reshape(1, -1), W1)
    p = _spmm(edata, adjt, h2, 128)
    h2 = _mm_fused(p[0], p[1], b1.reshape(1, -1), W2)
    p = _spmm(edata, adjt, h2, 128)
    W_out_p = jnp.zeros((W_out.shape[0], 128), jnp.float32).at[:, :W_out.shape[1]].set(W_out)
    h2 = _mm_fused(p[0], p[1], b2.reshape(1, -1), W_out_p)
    p = _spmm(edata, adjt, h2, 128)
    return _bias_add(p[0], p[1], b_out.reshape(1, -1))
